# Initial kernel scaffold; baseline (speedup 1.0000x reference)
import jax
import jax.numpy as jnp
from jax.experimental import pallas as pl


def kernel(x, edge_index, W, att_src, att_dst, bias):
    raise NotImplementedError("write your pallas kernel here")


# speedup vs baseline: 25.3204x; 25.3204x over previous
import functools

import jax
import jax.numpy as jnp
from jax import lax
from jax.experimental import pallas as pl
from jax.experimental.pallas import tpu as pltpu
from jax.experimental.pallas import tpu_sc as plsc

N = 10000
E = 320000
D = 128
NEG_SLOPE = 0.2

NUM_CORES = 2
NUM_SUBCORES = 16
NUM_WORKERS = NUM_CORES * NUM_SUBCORES
CHUNK = 128
NUM_CHUNKS = E // CHUNK
CHUNKS_PER_WORKER = -(-NUM_CHUNKS // NUM_WORKERS)
ROWS_PER_SUBCORE = 624
TAIL_START = ROWS_PER_SUBCORE * NUM_SUBCORES
TAIL_ROWS = N - TAIL_START


def _tc_prep_body(x_ref, w_ref, asrc_ref, adst_ref, h_ref, s_ref, d_ref):
    h = jnp.dot(x_ref[...], w_ref[...], preferred_element_type=jnp.float32)
    h_ref[...] = h
    s_ref[...] = jnp.dot(h, asrc_ref[...], preferred_element_type=jnp.float32)
    d_ref[...] = jnp.dot(h, adst_ref[...], preferred_element_type=jnp.float32)


def _tc_prep(x, W, att_src, att_dst):
    blk = 1000
    grid = N // blk
    return pl.pallas_call(
        _tc_prep_body,
        grid=(grid,),
        in_specs=[
            pl.BlockSpec((blk, D), lambda i: (i, 0)),
            pl.BlockSpec((D, D), lambda i: (0, 0)),
            pl.BlockSpec((D, 1), lambda i: (0, 0)),
            pl.BlockSpec((D, 1), lambda i: (0, 0)),
        ],
        out_specs=[
            pl.BlockSpec((blk, D), lambda i: (i, 0)),
            pl.BlockSpec((blk, 1), lambda i: (i, 0)),
            pl.BlockSpec((blk, 1), lambda i: (i, 0)),
        ],
        out_shape=[
            jax.ShapeDtypeStruct((N, D), jnp.float32),
            jax.ShapeDtypeStruct((N, 1), jnp.float32),
            jax.ShapeDtypeStruct((N, 1), jnp.float32),
        ],
    )(x, W, att_src.reshape(D, 1), att_dst.reshape(D, 1))


ZCHUNK = 48


def _sc_body(h_hbm, src_hbm, dst_hbm, as_hbm, ad_hbm,
             acc_out, den_out,
             as_v, ad_v, si_v, di_v, p_v, rows_v, stage1_v, acc_sp, den_sp,
             sem):
    c = lax.axis_index("c")
    s = lax.axis_index("s")
    wid = s * NUM_CORES + c

    pltpu.sync_copy(as_hbm, as_v)
    pltpu.sync_copy(ad_hbm, ad_v)

    def zero_row(j, carry2):
        for k in range(D // 16):
            rows_v[j, pl.ds(k * 16, 16)] = jnp.zeros((16,), jnp.float32)
        return carry2

    lax.fori_loop(0, CHUNK, zero_row, 0)

    def zero_s1(j, carry2):
        stage1_v[pl.ds(j * 16, 16)] = jnp.zeros((16,), jnp.float32)
        return carry2

    lax.fori_loop(0, ROWS_PER_SUBCORE // 16, zero_s1, 0)

    r0 = s * ROWS_PER_SUBCORE

    def zero_acc(t, carry2):
        pltpu.sync_copy(rows_v.at[pl.ds(0, ZCHUNK)],
                        acc_sp.at[pl.ds(r0 + t * ZCHUNK, ZCHUNK)])
        return carry2

    lax.fori_loop(0, ROWS_PER_SUBCORE // ZCHUNK, zero_acc, 0)
    pltpu.sync_copy(stage1_v, den_sp.at[pl.ds(r0, ROWS_PER_SUBCORE)])

    @pl.when(s == 0)
    def _zero_tail():
        pltpu.sync_copy(rows_v.at[pl.ds(0, TAIL_ROWS)],
                        acc_sp.at[pl.ds(TAIL_START, TAIL_ROWS)])
        pltpu.sync_copy(stage1_v.at[pl.ds(0, TAIL_ROWS)],
                        den_sp.at[pl.ds(TAIL_START, TAIL_ROWS)])

    plsc.subcore_barrier()

    def chunk_body(i, carry):
        cid = wid + NUM_WORKERS * i

        @pl.when(cid < NUM_CHUNKS)
        def _():
            base = cid * CHUNK
            pltpu.sync_copy(src_hbm.at[pl.ds(base, CHUNK)], si_v)
            pltpu.sync_copy(dst_hbm.at[pl.ds(base, CHUNK)], di_v)
            gather = pltpu.async_copy(h_hbm.at[si_v], rows_v, sem)
            for j in range(CHUNK // 16):
                sl = pl.ds(j * 16, 16)
                logit = (plsc.load_gather(as_v, [si_v[sl]])
                         + plsc.load_gather(ad_v, [di_v[sl]]))
                e = jnp.maximum(logit, logit * NEG_SLOPE)
                p_v[sl] = jnp.exp(e)
            pltpu.sync_copy(p_v, den_sp.at[di_v], add=True)
            gather.wait()

            def scale_row(j, carry2):
                pj = plsc.load_gather(p_v, [jnp.full((16,), j, jnp.int32)])
                for k in range(D // 16):
                    slk = pl.ds(k * 16, 16)
                    rows_v[j, slk] = rows_v[j, slk] * pj
                return carry2

            lax.fori_loop(0, CHUNK, scale_row, 0)
            pltpu.sync_copy(rows_v, acc_sp.at[di_v], add=True)

        return carry

    lax.fori_loop(0, CHUNKS_PER_WORKER, chunk_body, 0)
    plsc.subcore_barrier()

    def pub_acc(t, carry2):
        base = r0 + t * ZCHUNK
        pltpu.sync_copy(acc_sp.at[pl.ds(base, ZCHUNK)],
                        rows_v.at[pl.ds(0, ZCHUNK)])
        pltpu.sync_copy(rows_v.at[pl.ds(0, ZCHUNK)],
                        acc_out.at[c, pl.ds(base, ZCHUNK)])
        return carry2

    lax.fori_loop(0, ROWS_PER_SUBCORE // ZCHUNK, pub_acc, 0)
    pltpu.sync_copy(den_sp.at[pl.ds(r0, ROWS_PER_SUBCORE)], stage1_v)
    pltpu.sync_copy(stage1_v, den_out.at[pl.ds(c * N + r0, ROWS_PER_SUBCORE)])

    @pl.when(s == 0)
    def _publish_tail():
        pltpu.sync_copy(acc_sp.at[pl.ds(TAIL_START, TAIL_ROWS)],
                        rows_v.at[pl.ds(0, TAIL_ROWS)])
        pltpu.sync_copy(rows_v.at[pl.ds(0, TAIL_ROWS)],
                        acc_out.at[c, pl.ds(TAIL_START, TAIL_ROWS)])
        pltpu.sync_copy(den_sp.at[pl.ds(TAIL_START, TAIL_ROWS)],
                        stage1_v.at[pl.ds(0, TAIL_ROWS)])
        pltpu.sync_copy(stage1_v.at[pl.ds(0, TAIL_ROWS)],
                        den_out.at[pl.ds(c * N + TAIL_START, TAIL_ROWS)])


def _sc_edge_pass(h, src, dst, a_s, a_d):
    mesh = plsc.VectorSubcoreMesh(core_axis_name="c", subcore_axis_name="s")
    f = functools.partial(
        pl.kernel,
        mesh=mesh,
        compiler_params=pltpu.CompilerParams(needs_layout_passes=False),
        out_type=[
            jax.ShapeDtypeStruct((NUM_CORES, N, D), jnp.float32),
            jax.ShapeDtypeStruct((NUM_CORES * N,), jnp.float32),
        ],
        scratch_types=[
            pltpu.VMEM((N,), jnp.float32),
            pltpu.VMEM((N,), jnp.float32),
            pltpu.VMEM((CHUNK,), jnp.int32),
            pltpu.VMEM((CHUNK,), jnp.int32),
            pltpu.VMEM((CHUNK,), jnp.float32),
            pltpu.VMEM((CHUNK, D), jnp.float32),
            pltpu.VMEM((ROWS_PER_SUBCORE,), jnp.float32),
            pltpu.VMEM_SHARED((N, D), jnp.float32),
            pltpu.VMEM_SHARED((N,), jnp.float32),
            pltpu.SemaphoreType.DMA,
        ],
    )(_sc_body)
    return f(h, src, dst, a_s, a_d)


def _tc_combine_body(acc_ref, den_ref, bias_ref, out_ref):
    num = acc_ref[0] + acc_ref[1]
    den = den_ref[0] + den_ref[1] + 1e-16
    out_ref[...] = num / den + bias_ref[...]


def _tc_combine(acc, den, bias):
    blk = 1000
    grid = N // blk
    return pl.pallas_call(
        _tc_combine_body,
        grid=(grid,),
        in_specs=[
            pl.BlockSpec((NUM_CORES, blk, D), lambda i: (0, i, 0)),
            pl.BlockSpec((NUM_CORES, blk, 1), lambda i: (0, i, 0)),
            pl.BlockSpec((1, D), lambda i: (0, 0)),
        ],
        out_specs=pl.BlockSpec((blk, D), lambda i: (i, 0)),
        out_shape=jax.ShapeDtypeStruct((N, D), jnp.float32),
    )(acc, den, bias.reshape(1, D))


def kernel(x, edge_index, W, att_src, att_dst, bias):
    src = edge_index[0]
    dst = edge_index[1]
    h, a_s, a_d = _tc_prep(x, W, att_src, att_dst)
    acc, den = _sc_edge_pass(h, src, dst, a_s.reshape(N), a_d.reshape(N))
    return _tc_combine(acc, den.reshape(NUM_CORES, N, 1), bias)


# speedup vs baseline: 44.6185x; 1.7622x over previous
import functools

import jax
import jax.numpy as jnp
from jax import lax
from jax.experimental import pallas as pl
from jax.experimental.pallas import tpu as pltpu
from jax.experimental.pallas import tpu_sc as plsc

N = 10000
E = 320000
D = 128
NEG_SLOPE = 0.2

NUM_CORES = 2
NUM_SUBCORES = 16
NUM_WORKERS = NUM_CORES * NUM_SUBCORES
CHUNK = 128
NUM_CHUNKS = E // CHUNK
CHUNKS_PER_WORKER = -(-NUM_CHUNKS // NUM_WORKERS)
ROWS_PER_SUBCORE = 624
TAIL_START = ROWS_PER_SUBCORE * NUM_SUBCORES
TAIL_ROWS = N - TAIL_START


def _tc_prep_body(x_ref, w_ref, asrc_ref, adst_ref, h_ref, s_ref, d_ref):
    h = jnp.dot(x_ref[...], w_ref[...], preferred_element_type=jnp.float32)
    h_ref[...] = h
    s_ref[...] = jnp.dot(h, asrc_ref[...], preferred_element_type=jnp.float32)
    d_ref[...] = jnp.dot(h, adst_ref[...], preferred_element_type=jnp.float32)


def _tc_prep(x, W, att_src, att_dst):
    blk = 1000
    grid = N // blk
    return pl.pallas_call(
        _tc_prep_body,
        grid=(grid,),
        in_specs=[
            pl.BlockSpec((blk, D), lambda i: (i, 0)),
            pl.BlockSpec((D, D), lambda i: (0, 0)),
            pl.BlockSpec((D, 1), lambda i: (0, 0)),
            pl.BlockSpec((D, 1), lambda i: (0, 0)),
        ],
        out_specs=[
            pl.BlockSpec((blk, D), lambda i: (i, 0)),
            pl.BlockSpec((blk, 1), lambda i: (i, 0)),
            pl.BlockSpec((blk, 1), lambda i: (i, 0)),
        ],
        out_shape=[
            jax.ShapeDtypeStruct((N, D), jnp.float32),
            jax.ShapeDtypeStruct((N, 1), jnp.float32),
            jax.ShapeDtypeStruct((N, 1), jnp.float32),
        ],
    )(x, W, att_src.reshape(D, 1), att_dst.reshape(D, 1))


ZCHUNK = 48


def _sc_body(h_hbm, src_hbm, dst_hbm, as_hbm, ad_hbm,
             acc_out, den_out,
             si_v, di_v, p_v, ag_v, dg_v, rows_v, stage1_v, acc_sp, den_sp,
             gsem, ssem, isem, lsem):
    c = lax.axis_index("c")
    s = lax.axis_index("s")
    wid = s * NUM_CORES + c

    def zero_row(j, carry2):
        for k in range(D // 16):
            rows_v[0, j, pl.ds(k * 16, 16)] = jnp.zeros((16,), jnp.float32)
        return carry2

    lax.fori_loop(0, ZCHUNK, zero_row, 0)

    def zero_s1(j, carry2):
        stage1_v[pl.ds(j * 16, 16)] = jnp.zeros((16,), jnp.float32)
        return carry2

    lax.fori_loop(0, ROWS_PER_SUBCORE // 16, zero_s1, 0)

    r0 = s * ROWS_PER_SUBCORE

    def zero_acc(t, carry2):
        pltpu.sync_copy(rows_v.at[0, pl.ds(0, ZCHUNK)],
                        acc_sp.at[pl.ds(r0 + t * ZCHUNK, ZCHUNK)])
        return carry2

    lax.fori_loop(0, ROWS_PER_SUBCORE // ZCHUNK, zero_acc, 0)
    pltpu.sync_copy(stage1_v, den_sp.at[pl.ds(r0, ROWS_PER_SUBCORE)])

    @pl.when(s == 0)
    def _zero_tail():
        pltpu.sync_copy(rows_v.at[0, pl.ds(0, TAIL_ROWS)],
                        acc_sp.at[pl.ds(TAIL_START, TAIL_ROWS)])
        pltpu.sync_copy(stage1_v.at[pl.ds(0, TAIL_ROWS)],
                        den_sp.at[pl.ds(TAIL_START, TAIL_ROWS)])

    plsc.subcore_barrier()

    base_chunks = NUM_CHUNKS // NUM_WORKERS
    extra = NUM_CHUNKS - base_chunks * NUM_WORKERS
    my_n = base_chunks + jnp.where(wid < extra, 1, 0)
    my_start = wid * base_chunks + jnp.minimum(wid, extra)

    def start_load_idx(i, t):
        base = (my_start + i) * CHUNK
        pltpu.async_copy(src_hbm.at[pl.ds(base, CHUNK)], si_v.at[t], isem)
        pltpu.async_copy(dst_hbm.at[pl.ds(base, CHUNK)], di_v.at[t], isem)

    def wait_load_idx(t):
        pltpu.make_async_copy(src_hbm.at[pl.ds(0, CHUNK)], si_v.at[t],
                              isem).wait()
        pltpu.make_async_copy(dst_hbm.at[pl.ds(0, CHUNK)], di_v.at[t],
                              isem).wait()

    def start_gather(b, t):
        pltpu.async_copy(h_hbm.at[si_v.at[t]], rows_v.at[b], gsem)

    def wait_gather(b, t):
        pltpu.make_async_copy(h_hbm.at[si_v.at[t]], rows_v.at[b],
                              gsem).wait()

    def start_logit_gather(b, t):
        pltpu.async_copy(as_hbm.at[si_v.at[t]], ag_v.at[b], lsem)
        pltpu.async_copy(ad_hbm.at[di_v.at[t]], dg_v.at[b], lsem)

    def wait_logit_gather(b, t):
        pltpu.make_async_copy(as_hbm.at[si_v.at[t]], ag_v.at[b], lsem).wait()
        pltpu.make_async_copy(ad_hbm.at[di_v.at[t]], dg_v.at[b], lsem).wait()

    def compute_p(b, t):
        wait_logit_gather(b, t)
        for j in range(CHUNK // 16):
            sl = pl.ds(j * 16, 16)
            logit = ag_v[b, sl] + dg_v[b, sl]
            e = jnp.maximum(logit, logit * NEG_SLOPE)
            p_v[b, sl] = jnp.exp(e)
        pltpu.sync_copy(p_v.at[b], den_sp.at[di_v.at[t]], add=True)

    def scale_rows(b):
        def scale4(j, carry2):
            for r in range(4):
                row = j * 4 + r
                pj = plsc.load_gather(
                    p_v.at[b], [jnp.full((16,), row, jnp.int32)])
                for k in range(D // 16):
                    slk = pl.ds(k * 16, 16)
                    rows_v[b, row, slk] = rows_v[b, row, slk] * pj
            return carry2

        lax.fori_loop(0, CHUNK // 4, scale4, 0)

    def start_scatter(b, t):
        pltpu.async_copy(rows_v.at[b], acc_sp.at[di_v.at[t]], ssem, add=True)

    def wait_scatter(b, t):
        pltpu.make_async_copy(rows_v.at[b], acc_sp.at[di_v.at[t]],
                              ssem).wait()

    start_load_idx(0, 0)
    wait_load_idx(0)
    start_logit_gather(0, 0)
    start_gather(0, 0)
    start_load_idx(1, 1)

    def chunk_body(i, carry):
        b = lax.rem(i, 2)
        nb = 1 - b
        t = lax.rem(i, 3)
        tn = lax.rem(i + 1, 3)
        tp = lax.rem(i + 2, 3)

        compute_p(b, t)
        wait_gather(b, t)

        @pl.when(i >= 1)
        def _wait_prev_scatter():
            wait_scatter(nb, tp)

        @pl.when(i + 2 < my_n)
        def _prefetch_idx():
            start_load_idx(i + 2, tp)

        wait_load_idx(tn)
        start_logit_gather(nb, tn)
        start_gather(nb, tn)
        scale_rows(b)
        start_scatter(b, t)
        return carry

    lax.fori_loop(0, my_n - 1, chunk_body, 0)

    last = my_n - 1
    lb = lax.rem(last, 2)
    lt = lax.rem(last, 3)
    ltp = lax.rem(last + 2, 3)
    compute_p(lb, lt)
    wait_gather(lb, lt)
    wait_scatter(1 - lb, ltp)
    scale_rows(lb)
    pltpu.sync_copy(rows_v.at[lb], acc_sp.at[di_v.at[lt]], add=True)
    plsc.subcore_barrier()

    def pub_acc(t, carry2):
        base = r0 + t * ZCHUNK
        pltpu.sync_copy(acc_sp.at[pl.ds(base, ZCHUNK)],
                        rows_v.at[0, pl.ds(0, ZCHUNK)])
        pltpu.sync_copy(rows_v.at[0, pl.ds(0, ZCHUNK)],
                        acc_out.at[c, pl.ds(base, ZCHUNK)])
        return carry2

    lax.fori_loop(0, ROWS_PER_SUBCORE // ZCHUNK, pub_acc, 0)
    pltpu.sync_copy(den_sp.at[pl.ds(r0, ROWS_PER_SUBCORE)], stage1_v)
    pltpu.sync_copy(stage1_v, den_out.at[pl.ds(c * N + r0, ROWS_PER_SUBCORE)])

    @pl.when(s == 0)
    def _publish_tail():
        pltpu.sync_copy(acc_sp.at[pl.ds(TAIL_START, TAIL_ROWS)],
                        rows_v.at[0, pl.ds(0, TAIL_ROWS)])
        pltpu.sync_copy(rows_v.at[0, pl.ds(0, TAIL_ROWS)],
                        acc_out.at[c, pl.ds(TAIL_START, TAIL_ROWS)])
        pltpu.sync_copy(den_sp.at[pl.ds(TAIL_START, TAIL_ROWS)],
                        stage1_v.at[pl.ds(0, TAIL_ROWS)])
        pltpu.sync_copy(stage1_v.at[pl.ds(0, TAIL_ROWS)],
                        den_out.at[pl.ds(c * N + TAIL_START, TAIL_ROWS)])


def _sc_edge_pass(h, src, dst, a_s, a_d):
    mesh = plsc.VectorSubcoreMesh(core_axis_name="c", subcore_axis_name="s")
    f = functools.partial(
        pl.kernel,
        mesh=mesh,
        compiler_params=pltpu.CompilerParams(needs_layout_passes=False),
        out_type=[
            jax.ShapeDtypeStruct((NUM_CORES, N, D), jnp.float32),
            jax.ShapeDtypeStruct((NUM_CORES * N,), jnp.float32),
        ],
        scratch_types=[
            pltpu.VMEM((3, CHUNK), jnp.int32),
            pltpu.VMEM((3, CHUNK), jnp.int32),
            pltpu.VMEM((2, CHUNK), jnp.float32),
            pltpu.VMEM((2, CHUNK), jnp.float32),
            pltpu.VMEM((2, CHUNK), jnp.float32),
            pltpu.VMEM((2, CHUNK, D), jnp.float32),
            pltpu.VMEM((ROWS_PER_SUBCORE,), jnp.float32),
            pltpu.VMEM_SHARED((N, D), jnp.float32),
            pltpu.VMEM_SHARED((N,), jnp.float32),
            pltpu.SemaphoreType.DMA,
            pltpu.SemaphoreType.DMA,
            pltpu.SemaphoreType.DMA,
            pltpu.SemaphoreType.DMA,
        ],
    )(_sc_body)
    return f(h, src, dst, a_s, a_d)


def _tc_combine_body(acc_ref, den_ref, bias_ref, out_ref):
    num = acc_ref[0] + acc_ref[1]
    den = den_ref[0] + den_ref[1] + 1e-16
    out_ref[...] = num / den + bias_ref[...]


def _tc_combine(acc, den, bias):
    blk = 1000
    grid = N // blk
    return pl.pallas_call(
        _tc_combine_body,
        grid=(grid,),
        in_specs=[
            pl.BlockSpec((NUM_CORES, blk, D), lambda i: (0, i, 0)),
            pl.BlockSpec((NUM_CORES, blk, 1), lambda i: (0, i, 0)),
            pl.BlockSpec((1, D), lambda i: (0, 0)),
        ],
        out_specs=pl.BlockSpec((blk, D), lambda i: (i, 0)),
        out_shape=jax.ShapeDtypeStruct((N, D), jnp.float32),
    )(acc, den, bias.reshape(1, D))


def kernel(x, edge_index, W, att_src, att_dst, bias):
    src = edge_index[0]
    dst = edge_index[1]
    h, a_s, a_d = _tc_prep(x, W, att_src, att_dst)
    acc, den = _sc_edge_pass(h, src, dst, a_s.reshape(N), a_d.reshape(N))
    return _tc_combine(acc, den.reshape(NUM_CORES, N, 1), bias)
